# baseline (speedup 1.0000x reference)
import functools

import jax
import jax.numpy as jnp
from jax import lax
from jax.experimental import pallas as pl
from jax.experimental.pallas import tpu as pltpu
from jax.experimental.pallas import tpu_sc as plsc

B, C, H, W = 16, 256, 64, 64
HW = H * W
N = B * C * HW
NC, NS, L = 2, 16, 16
NW = NC * NS
CPW = C // NW
ROWS = B * CPW
NBUF = 4
VPR = HW // L


def _body(x_hbm, tab_hbm, out_hbm, tab_v, pos_v, xin, xout, sin, sout):
    wid = lax.axis_index("s") * NC + lax.axis_index("c")
    c_base = wid * CPW

    pltpu.sync_copy(tab_hbm, tab_v)

    lane = jnp.arange(L, dtype=jnp.int32)

    for r in range(CPW):
        c = c_base + r
        cvec = jnp.full((L,), c, dtype=jnp.int32)
        is_col = cvec < 128

        @pl.loop(0, VPR)
        def _pos(v, r=r, cvec=cvec, is_col=is_col):
            hw = v * L + lane
            idx0 = jnp.where(is_col, hw & 63, 64 + (hw >> 6))
            idx1 = jnp.where(is_col, cvec, cvec - 128)
            val = plsc.load_gather(tab_v, [idx0, idx1])
            pos_v[pl.ds(r * HW + v * L, L)] = val

    def row_off(k):
        return ((k >> 3) * C + c_base + (k & 7)) * HW

    def start_in(s, k):
        pltpu.make_async_copy(
            x_hbm.at[pl.ds(row_off(k), HW)], xin[s], sin.at[s]
        ).start()

    def wait_in(s, k):
        pltpu.make_async_copy(
            x_hbm.at[pl.ds(row_off(k), HW)], xin[s], sin.at[s]
        ).wait()

    def start_out(s, k):
        pltpu.make_async_copy(
            xout[s], out_hbm.at[pl.ds(row_off(k), HW)], sout.at[s]
        ).start()

    def wait_out(s, k):
        pltpu.make_async_copy(
            xout[s], out_hbm.at[pl.ds(row_off(k), HW)], sout.at[s]
        ).wait()

    for s in range(NBUF):
        start_in(s, s)

    @pl.loop(0, ROWS // NBUF)
    def _grp(g):
        k0 = g * NBUF
        for s in range(NBUF):
            k = k0 + s
            wait_in(s, k)

            @pl.when(k >= NBUF)
            def _():
                wait_out(s, k - NBUF)

            pos_off = (k & 7) * HW

            @pl.loop(0, VPR, unroll=8)
            def _add(v, s=s, pos_off=pos_off):
                sl = pl.ds(v * L, L)
                xout[s][sl] = xin[s][sl] + pos_v[pl.ds(pos_off + v * L, L)]

            start_out(s, k)

            @pl.when(k + NBUF < ROWS)
            def _():
                start_in(s, k + NBUF)

    for s in range(NBUF):
        wait_out(s, ROWS - NBUF + s)


@functools.partial(jax.jit, static_argnames=())
def _run(xf, table):
    kern = pl.kernel(
        _body,
        out_type=jax.ShapeDtypeStruct((N,), jnp.float32),
        mesh=plsc.VectorSubcoreMesh(core_axis_name="c", subcore_axis_name="s"),
        compiler_params=pltpu.CompilerParams(needs_layout_passes=False),
        scratch_types=[
            pltpu.VMEM((128, 128), jnp.float32),
            pltpu.VMEM((CPW * HW,), jnp.float32),
            [pltpu.VMEM((HW,), jnp.float32) for _ in range(NBUF)],
            [pltpu.VMEM((HW,), jnp.float32) for _ in range(NBUF)],
            pltpu.SemaphoreType.DMA((NBUF,)),
            pltpu.SemaphoreType.DMA((NBUF,)),
        ],
    )
    return kern(xf, table)


def kernel(x, row_embed, col_embed):
    table = jnp.concatenate([col_embed, row_embed], axis=0)
    out = _run(x.reshape(N), table)
    return out.reshape(B, C, H, W)


# speedup vs baseline: 1.5654x; 1.5654x over previous
import jax
import jax.numpy as jnp
from jax import lax
from jax.experimental import pallas as pl
from jax.experimental.pallas import tpu as pltpu
from jax.experimental.pallas import tpu_sc as plsc

B, C, H, W = 16, 256, 64, 64
HW = H * W
NC, NS, L = 2, 16, 16
NW = NC * NS
CPW = C // NW
VPR = HW // L
NBUF = 2
GROUPS = B


def _body(x_hbm, tab_hbm, out_hbm, tab_v, pos_v, xg0, xg1, sin, sout):
    wid = lax.axis_index("s") * NC + lax.axis_index("c")
    c_base = wid * CPW
    xg = (xg0, xg1)

    pltpu.sync_copy(tab_hbm, tab_v)

    lane = jnp.arange(L, dtype=jnp.int32)

    for r in range(CPW):
        c = c_base + r
        cvec = jnp.full((L,), c, dtype=jnp.int32)
        is_col = cvec < 128

        @pl.loop(0, VPR)
        def _pos(v, r=r, cvec=cvec, is_col=is_col):
            hw = v * L + lane
            idx = jnp.where(
                is_col,
                (hw & 63) * 128 + cvec,
                8192 + (hw >> 6) * 128 + cvec - 128,
            )
            val = plsc.load_gather(tab_v, [idx])
            pos_v[r, pl.ds(v * L, L)] = val

    def start_in(s, g):
        pltpu.make_async_copy(
            x_hbm.at[g, pl.ds(c_base, CPW), :], xg[s], sin.at[s]
        ).start()

    def wait_in(s, g):
        pltpu.make_async_copy(
            x_hbm.at[g, pl.ds(c_base, CPW), :], xg[s], sin.at[s]
        ).wait()

    def start_out(s, g):
        pltpu.make_async_copy(
            xg[s], out_hbm.at[g, pl.ds(c_base, CPW), :], sout.at[s]
        ).start()

    def wait_out(s, g):
        pltpu.make_async_copy(
            xg[s], out_hbm.at[g, pl.ds(c_base, CPW), :], sout.at[s]
        ).wait()

    for s in range(NBUF):
        start_in(s, s)

    @pl.loop(0, GROUPS // NBUF)
    def _pair(t):
        for s in range(NBUF):
            g = t * NBUF + s
            wait_in(s, g)

            for r in range(CPW):

                @pl.loop(0, VPR, unroll=8)
                def _add(v, s=s, r=r):
                    sl = pl.ds(v * L, L)
                    xg[s][r, sl] = xg[s][r, sl] + pos_v[r, sl]

            start_out(s, g)

            @pl.when(g + NBUF < GROUPS)
            def _(s=s, g=g):
                wait_out(s, g)
                start_in(s, g + NBUF)

    for s in range(NBUF):
        wait_out(s, GROUPS - NBUF + s)


def _run(x3, table):
    kern = pl.kernel(
        _body,
        out_type=jax.ShapeDtypeStruct((B, C, HW), jnp.float32),
        mesh=plsc.VectorSubcoreMesh(core_axis_name="c", subcore_axis_name="s"),
        compiler_params=pltpu.CompilerParams(
            needs_layout_passes=False, use_tc_tiling_on_sc=True
        ),
        scratch_types=[
            pltpu.VMEM((128 * 128,), jnp.float32),
            pltpu.VMEM((CPW, HW), jnp.float32),
            pltpu.VMEM((CPW, HW), jnp.float32),
            pltpu.VMEM((CPW, HW), jnp.float32),
            pltpu.SemaphoreType.DMA((NBUF,)),
            pltpu.SemaphoreType.DMA((NBUF,)),
        ],
    )
    return kern(x3, table)


def kernel(x, row_embed, col_embed):
    table = jnp.concatenate([col_embed, row_embed], axis=0).reshape(-1)
    out = _run(x.reshape(B, C, HW), table)
    return out.reshape(B, C, H, W)


# speedup vs baseline: 2.4126x; 1.5411x over previous
import jax
import jax.numpy as jnp
from jax import lax
from jax.experimental import pallas as pl
from jax.experimental.pallas import tpu as pltpu
from jax.experimental.pallas import tpu_sc as plsc

B, C, H, W = 16, 256, 64, 64
HW = H * W
NC, NS, L = 2, 16, 16
NW = NC * NS
CPW = C // NW
VPR = HW // L
NBUF = 2
GROUPS = B


def _body(x_hbm, tab_hbm, out_hbm, tab_v, pos_v, xg0, xg1, sin, sout):
    wid = lax.axis_index("s") * NC + lax.axis_index("c")
    c_base = wid * CPW
    xg = (xg0, xg1)

    pltpu.sync_copy(tab_hbm, tab_v)

    lane = jnp.arange(L, dtype=jnp.int32)

    for r in range(CPW):
        c = c_base + r
        cvec = jnp.full((L,), c, dtype=jnp.int32)
        is_col = cvec < 128

        @plsc.parallel_loop(0, VPR, unroll=4)
        def _pos(v, r=r, cvec=cvec, is_col=is_col):
            hw = v * L + lane
            idx = jnp.where(
                is_col,
                (hw & 63) * 128 + cvec,
                8192 + (hw >> 6) * 128 + cvec - 128,
            )
            val = plsc.load_gather(tab_v, [idx])
            pos_v[r, pl.ds(v * L, L)] = val

    def start_in(s, g):
        pltpu.make_async_copy(
            x_hbm.at[g, pl.ds(c_base, CPW), :], xg[s], sin.at[s]
        ).start()

    def wait_in(s, g):
        pltpu.make_async_copy(
            x_hbm.at[g, pl.ds(c_base, CPW), :], xg[s], sin.at[s]
        ).wait()

    def start_out(s, g):
        pltpu.make_async_copy(
            xg[s], out_hbm.at[g, pl.ds(c_base, CPW), :], sout.at[s]
        ).start()

    def wait_out(s, g):
        pltpu.make_async_copy(
            xg[s], out_hbm.at[g, pl.ds(c_base, CPW), :], sout.at[s]
        ).wait()

    for s in range(NBUF):
        start_in(s, s)

    @pl.loop(0, GROUPS // NBUF)
    def _pair(t):
        for s in range(NBUF):
            g = t * NBUF + s
            wait_in(s, g)

            for r in range(CPW):

                @plsc.parallel_loop(0, VPR, unroll=8)
                def _add(v, s=s, r=r):
                    sl = pl.ds(v * L, L)
                    xg[s][r, sl] = xg[s][r, sl] + pos_v[r, sl]

            start_out(s, g)

            @pl.when(g + NBUF < GROUPS)
            def _(s=s, g=g):
                wait_out(s, g)
                start_in(s, g + NBUF)

    for s in range(NBUF):
        wait_out(s, GROUPS - NBUF + s)


def _run(x3, table):
    kern = pl.kernel(
        _body,
        out_type=jax.ShapeDtypeStruct((B, C, HW), jnp.float32),
        mesh=plsc.VectorSubcoreMesh(core_axis_name="c", subcore_axis_name="s"),
        compiler_params=pltpu.CompilerParams(
            needs_layout_passes=False, use_tc_tiling_on_sc=True
        ),
        scratch_types=[
            pltpu.VMEM((128 * 128,), jnp.float32),
            pltpu.VMEM((CPW, HW), jnp.float32),
            pltpu.VMEM((CPW, HW), jnp.float32),
            pltpu.VMEM((CPW, HW), jnp.float32),
            pltpu.SemaphoreType.DMA((NBUF,)),
            pltpu.SemaphoreType.DMA((NBUF,)),
        ],
    )
    return kern(x3, table)


def kernel(x, row_embed, col_embed):
    table = jnp.concatenate([col_embed, row_embed], axis=0).reshape(-1)
    out = _run(x.reshape(B, C, HW), table)
    return out.reshape(B, C, H, W)
